# Initial kernel scaffold; baseline (speedup 1.0000x reference)
import functools

import jax
import jax.numpy as jnp
from jax import lax
from jax.experimental import pallas as pl
from jax.experimental.pallas import tpu as pltpu
from jax.experimental.pallas import tpu_sc as plsc

N_ATOMS = 100000
N_EDGES = 6400000
N_MOL = 1000

KE = 1.0
CUTON = 2.5
CUTOFF = 7.5
LR_CUTOFF = 10.0
KEHALF = KE / 2.0
CUTON16 = CUTON ** 16
CUT_RCONSTANT = LR_CUTOFF ** 15 / (LR_CUTOFF ** 16 + CUTON16) ** (17.0 / 16.0)
CUT_CONSTANT = (1.0 / (CUTON16 + LR_CUTOFF ** 16) ** (1.0 / 16.0)
                + LR_CUTOFF ** 16 / (LR_CUTOFF ** 16 + CUTON16) ** (17.0 / 16.0))


_RB = 1000
_FLAT_COLS = 384
_FLAT_ROWS = N_EDGES * 3 // _FLAT_COLS


def _g_body(r_ref, o_ref):
    r = r_ref[...]
    sq = r * r
    k = lax.broadcasted_iota(jnp.int32, (_FLAT_COLS, 128), 0)
    e = lax.broadcasted_iota(jnp.int32, (_FLAT_COLS, 128), 1)
    sel = (k // 3 == e).astype(jnp.float32)
    d2 = lax.dot_general(sq, sel, (((1,), (0,)), ((), ())),
                         preferred_element_type=jnp.float32)
    d = jnp.sqrt(d2)
    x = (d - CUTON) / (CUTOFF - CUTON)
    x3 = x * x * x
    x4 = x3 * x
    x5 = x4 * x
    sw = 1.0 - 6.0 * x5 + 15.0 * x4 - 10.0 * x3
    f = jnp.where(x <= 0.0, 1.0, jnp.where(x >= 1.0, 0.0, sw))
    coul = jnp.where(d < LR_CUTOFF,
                     1.0 / d + d / (LR_CUTOFF ** 2) - 2.0 / LR_CUTOFF, 0.0)
    t = d2 * d2
    t = t * t
    t = t * t
    t = t + CUTON16
    p = lax.rsqrt(t)
    p = jnp.sqrt(p)
    p = jnp.sqrt(p)
    p = jnp.sqrt(p)
    damped = p + (1.0 - f) * CUT_RCONSTANT * d - CUT_CONSTANT
    o_ref[...] = KEHALF * (f * damped + (1.0 - f) * coul)


def _g_factor(Rij):
    r2 = Rij.reshape(_FLAT_ROWS, _FLAT_COLS)
    out = pl.pallas_call(
        _g_body,
        grid=(_FLAT_ROWS // _RB,),
        in_specs=[pl.BlockSpec((_RB, _FLAT_COLS), lambda i: (i, 0))],
        out_specs=pl.BlockSpec((_RB, 128), lambda i: (i, 0)),
        out_shape=jax.ShapeDtypeStruct((_FLAT_ROWS, 128), jnp.float32),
    )(r2)
    return out.reshape(N_EDGES)


_NC, _NS, _L = 2, 16, 16
_NW = _NC * _NS
_EPW = N_EDGES // _NW
_CH = 2000
_NCHUNK = _EPW // _CH
_NVEC = _CH // _L
_MB = 1008
_QMASK = jnp.int32(-1024)
_MMASK = jnp.int32(1023)

_mesh = plsc.VectorSubcoreMesh(core_axis_name="c", subcore_axis_name="s",
                               num_cores=_NC, num_subcores=_NS)


@functools.partial(
    pl.kernel,
    out_type=jax.ShapeDtypeStruct((_NW, _MB), jnp.float32),
    mesh=_mesh,
    scratch_types=[
        pltpu.VMEM((N_ATOMS,), jnp.int32),
        pltpu.VMEM((_L, _MB), jnp.float32),
        pltpu.VMEM((2, _CH), jnp.int32),
        pltpu.VMEM((2, _CH), jnp.int32),
        pltpu.VMEM((2, _CH), jnp.float32),
        pltpu.SemaphoreType.DMA,
        pltpu.SemaphoreType.DMA,
    ],
)
def _sc_scatter(packed_hbm, idxi_hbm, idxj_hbm, g_hbm, out_hbm,
                tab, bins, bi, bj, bg, sem0, sem1):
    wid = lax.axis_index("s") * _NC + lax.axis_index("c")
    base = wid * _EPW
    sems = (sem0, sem1)

    pltpu.sync_copy(packed_hbm, tab)

    zz = jnp.zeros((_L,), jnp.float32)

    def zero_body(c, carry):
        s = pl.ds(c * _L, _L)
        for r in range(_L):
            bins[r, s] = zz
        return carry

    lax.fori_loop(0, _MB // _L, zero_body, None)

    rows = lax.broadcasted_iota(jnp.int32, (_L,), 0)

    def start(c, slot):
        off = base + c * _CH
        pltpu.async_copy(idxi_hbm.at[pl.ds(off, _CH)], bi.at[slot], sems[slot])
        pltpu.async_copy(idxj_hbm.at[pl.ds(off, _CH)], bj.at[slot], sems[slot])
        pltpu.async_copy(g_hbm.at[pl.ds(off, _CH)], bg.at[slot], sems[slot])

    def drain(slot):
        pltpu.make_async_copy(idxi_hbm.at[pl.ds(0, _CH)], bi.at[slot],
                              sems[slot]).wait()
        pltpu.make_async_copy(idxj_hbm.at[pl.ds(0, _CH)], bj.at[slot],
                              sems[slot]).wait()
        pltpu.make_async_copy(g_hbm.at[pl.ds(0, _CH)], bg.at[slot],
                              sems[slot]).wait()

    def compute(slot):
        def vec_body(k, carry):
            s = pl.ds(k * _L, _L)
            vi = bi[slot, s]
            vj = bj[slot, s]
            gv = bg[slot, s]
            pi = plsc.load_gather(tab, [vi])
            pj = plsc.load_gather(tab, [vj])
            qi = plsc.bitcast(pi & _QMASK, jnp.float32)
            qj = plsc.bitcast(pj & _QMASK, jnp.float32)
            m = pi & _MMASK
            plsc.addupdate_scatter(bins, [rows, m], qi * qj * gv)
            return carry

        lax.fori_loop(0, _NVEC, vec_body, None)

    start(0, 0)

    def ring_body(h, carry):
        c0 = 2 * h
        start(c0 + 1, 1)
        drain(0)
        compute(0)

        @pl.when(c0 + 2 < _NCHUNK)
        def _():
            start(c0 + 2, 0)

        drain(1)
        compute(1)
        return carry

    lax.fori_loop(0, _NCHUNK // 2, ring_body, None)

    def red_body(c, carry):
        s = pl.ds(c * _L, _L)
        acc = bins[0, s]
        for r in range(1, _L):
            acc = acc + bins[r, s]
        bins[0, s] = acc
        return carry

    lax.fori_loop(0, _MB // _L, red_body, None)

    pltpu.sync_copy(bins.at[0], out_hbm.at[wid])


def kernel(Z, partial_charges, Rij, idx_i, idx_j, idx_m):
    q = jnp.squeeze(partial_charges, -1)
    qbits = lax.bitcast_convert_type(q, jnp.int32)
    packed = ((qbits + 512) & _QMASK) | idx_m.astype(jnp.int32)
    g = _g_factor(Rij)
    partials = _sc_scatter(packed, idx_i.astype(jnp.int32),
                           idx_j.astype(jnp.int32), g)
    return jnp.sum(partials, axis=0)[:N_MOL]


# speedup vs baseline: 15.7493x; 15.7493x over previous
import functools

import jax
import jax.numpy as jnp
import numpy as np
from jax import lax
from jax.experimental import pallas as pl
from jax.experimental.pallas import tpu as pltpu
from jax.experimental.pallas import tpu_sc as plsc

N_ATOMS = 100000
N_EDGES = 6400000
N_MOL = 1000

KE = 1.0
CUTON = 2.5
CUTOFF = 7.5
LR_CUTOFF = 10.0
KEHALF = KE / 2.0
CUTON16 = CUTON ** 16
CUT_RCONSTANT = LR_CUTOFF ** 15 / (LR_CUTOFF ** 16 + CUTON16) ** (17.0 / 16.0)
CUT_CONSTANT = (1.0 / (CUTON16 + LR_CUTOFF ** 16) ** (1.0 / 16.0)
                + LR_CUTOFF ** 16 / (LR_CUTOFF ** 16 + CUTON16) ** (17.0 / 16.0))


_RB = 1000
_FLAT_COLS = 384
_FLAT_ROWS = N_EDGES * 3 // _FLAT_COLS


def _g_body(r_ref, o_ref):
    r = r_ref[...]
    sq = r * r
    k = lax.broadcasted_iota(jnp.int32, (_FLAT_COLS, 128), 0)
    e = lax.broadcasted_iota(jnp.int32, (_FLAT_COLS, 128), 1)
    sel = (k // 3 == e).astype(jnp.float32)
    d2 = lax.dot_general(sq, sel, (((1,), (0,)), ((), ())),
                         preferred_element_type=jnp.float32)
    d = jnp.sqrt(d2)
    x = (d - CUTON) / (CUTOFF - CUTON)
    x3 = x * x * x
    x4 = x3 * x
    x5 = x4 * x
    sw = 1.0 - 6.0 * x5 + 15.0 * x4 - 10.0 * x3
    f = jnp.where(x <= 0.0, 1.0, jnp.where(x >= 1.0, 0.0, sw))
    coul = jnp.where(d < LR_CUTOFF,
                     1.0 / d + d / (LR_CUTOFF ** 2) - 2.0 / LR_CUTOFF, 0.0)
    t = d2 * d2
    t = t * t
    t = t * t
    t = t + CUTON16
    p = lax.rsqrt(t)
    p = jnp.sqrt(p)
    p = jnp.sqrt(p)
    p = jnp.sqrt(p)
    damped = p + (1.0 - f) * CUT_RCONSTANT * d - CUT_CONSTANT
    o_ref[...] = KEHALF * (f * damped + (1.0 - f) * coul)


def _g_factor(Rij):
    r2 = Rij.reshape(_FLAT_ROWS, _FLAT_COLS)
    out = pl.pallas_call(
        _g_body,
        grid=(_FLAT_ROWS // _RB,),
        in_specs=[pl.BlockSpec((_RB, _FLAT_COLS), lambda i: (i, 0))],
        out_specs=pl.BlockSpec((_RB, 128), lambda i: (i, 0)),
        out_shape=jax.ShapeDtypeStruct((_FLAT_ROWS, 128), jnp.float32),
    )(r2)
    return out.reshape(N_EDGES)


_NC, _NS, _L = 2, 16, 16
_NW = _NC * _NS
_EPW = N_EDGES // _NW
_CH = 2000
_NCHUNK = _EPW // _CH
_NVEC = _CH // _L
_MB = 1008
_QMASK = np.int32(-1024)
_MMASK = np.int32(1023)

def _sc_scatter_body(packed_hbm, idxi_hbm, idxj_hbm, g_hbm, out_hbm,
                     tab, bins, bi0, bi1, bj0, bj1, bg0, bg1, sem0, sem1):
    wid = lax.axis_index("s") * _NC + lax.axis_index("c")
    base = wid * _EPW
    bi = (bi0, bi1)
    bj = (bj0, bj1)
    bg = (bg0, bg1)
    sems = (sem0, sem1)

    pltpu.sync_copy(packed_hbm, tab)

    zz = jnp.zeros((_L,), jnp.float32)

    def zero_body(c, carry):
        s = pl.ds(c * _L, _L)
        for r in range(_L):
            bins[r, s] = zz
        return carry

    lax.fori_loop(0, _MB // _L, zero_body, None)

    rows = lax.broadcasted_iota(jnp.int32, (_L,), 0)

    def start(c, slot):
        off = base + c * _CH
        pltpu.async_copy(idxi_hbm.at[pl.ds(off, _CH)], bi[slot], sems[slot])
        pltpu.async_copy(idxj_hbm.at[pl.ds(off, _CH)], bj[slot], sems[slot])
        pltpu.async_copy(g_hbm.at[pl.ds(off, _CH)], bg[slot], sems[slot])

    def drain(slot):
        pltpu.make_async_copy(idxi_hbm.at[pl.ds(0, _CH)], bi[slot],
                              sems[slot]).wait()
        pltpu.make_async_copy(idxj_hbm.at[pl.ds(0, _CH)], bj[slot],
                              sems[slot]).wait()
        pltpu.make_async_copy(g_hbm.at[pl.ds(0, _CH)], bg[slot],
                              sems[slot]).wait()

    def compute(slot):
        def vec_body(k, carry):
            s = pl.ds(k * _L, _L)
            vi = bi[slot][s]
            vj = bj[slot][s]
            gv = bg[slot][s]
            pi = plsc.load_gather(tab, [vi])
            pj = plsc.load_gather(tab, [vj])
            qi = plsc.bitcast(pi & _QMASK, jnp.float32)
            qj = plsc.bitcast(pj & _QMASK, jnp.float32)
            m = pi & _MMASK
            plsc.addupdate_scatter(bins, [rows, m], qi * qj * gv)
            return carry

        lax.fori_loop(0, _NVEC, vec_body, None)

    start(0, 0)

    def ring_body(h, carry):
        c0 = 2 * h
        start(c0 + 1, 1)
        drain(0)
        compute(0)

        @pl.when(c0 + 2 < _NCHUNK)
        def _():
            start(c0 + 2, 0)

        drain(1)
        compute(1)
        return carry

    lax.fori_loop(0, _NCHUNK // 2, ring_body, None)

    def red_body(c, carry):
        s = pl.ds(c * _L, _L)
        acc = bins[0, s]
        for r in range(1, _L):
            acc = acc + bins[r, s]
        bins[0, s] = acc
        return carry

    lax.fori_loop(0, _MB // _L, red_body, None)

    pltpu.sync_copy(bins.at[0], out_hbm.at[wid])


@functools.cache
def _sc_scatter_kernel():
    mesh = plsc.VectorSubcoreMesh(core_axis_name="c", subcore_axis_name="s",
                                  num_cores=_NC, num_subcores=_NS)
    return pl.kernel(
        _sc_scatter_body,
        out_type=jax.ShapeDtypeStruct((_NW, _MB), jnp.float32),
        mesh=mesh,
        compiler_params=pltpu.CompilerParams(needs_layout_passes=False),
        scratch_types=[
            pltpu.VMEM((N_ATOMS,), jnp.int32),
            pltpu.VMEM((_L, _MB), jnp.float32),
            pltpu.VMEM((_CH,), jnp.int32),
            pltpu.VMEM((_CH,), jnp.int32),
            pltpu.VMEM((_CH,), jnp.int32),
            pltpu.VMEM((_CH,), jnp.int32),
            pltpu.VMEM((_CH,), jnp.float32),
            pltpu.VMEM((_CH,), jnp.float32),
            pltpu.SemaphoreType.DMA,
            pltpu.SemaphoreType.DMA,
        ],
    )


def kernel(Z, partial_charges, Rij, idx_i, idx_j, idx_m):
    q = jnp.squeeze(partial_charges, -1)
    qbits = lax.bitcast_convert_type(q, jnp.int32)
    packed = ((qbits + 512) & _QMASK) | idx_m.astype(jnp.int32)
    g = _g_factor(Rij)
    partials = _sc_scatter_kernel()(packed, idx_i.astype(jnp.int32),
                                    idx_j.astype(jnp.int32), g)
    return jnp.sum(partials, axis=0)[:N_MOL]


# speedup vs baseline: 379.7529x; 24.1124x over previous
import functools

import jax
import jax.numpy as jnp
import numpy as np
from jax import lax
from jax.experimental import pallas as pl
from jax.experimental.pallas import tpu as pltpu
from jax.experimental.pallas import tpu_sc as plsc

N_ATOMS = 100000
N_EDGES = 6400000
N_MOL = 1000

CUTON = 2.5
CUTOFF = 7.5
LR_CUTOFF = 10.0
CUTON16 = CUTON ** 16
CUT_RCONSTANT = LR_CUTOFF ** 15 / (LR_CUTOFF ** 16 + CUTON16) ** (17.0 / 16.0)
CUT_CONSTANT = (1.0 / (CUTON16 + LR_CUTOFF ** 16) ** (1.0 / 16.0)
                + LR_CUTOFF ** 16 / (LR_CUTOFF ** 16 + CUTON16) ** (17.0 / 16.0))

_NSLICE = 5
_EPS = N_EDGES // _NSLICE


_RB = 1000
_SLICE_ROWS = _EPS // 128


def _g_body(x_ref, y_ref, z_ref, o_ref):
    x = x_ref[...]
    y = y_ref[...]
    z = z_ref[...]
    d2 = x * x + y * y + z * z
    d = jnp.sqrt(d2)
    u = (d - CUTON) * (1.0 / (CUTOFF - CUTON))
    u = jnp.clip(u, 0.0, 1.0)
    u3 = u * u * u
    f = 1.0 + u3 * (-10.0 + u * (15.0 - 6.0 * u))
    coul = jnp.where(d < LR_CUTOFF,
                     1.0 / d + d * (1.0 / LR_CUTOFF ** 2) - 2.0 / LR_CUTOFF,
                     0.0)
    t = d2 * d2
    t = t * t
    t = t * t + CUTON16
    p = lax.rsqrt(t)
    p = jnp.sqrt(p)
    p = jnp.sqrt(p)
    p = jnp.sqrt(p)
    damped = p + (1.0 - f) * CUT_RCONSTANT * d - CUT_CONSTANT
    o_ref[...] = f * (damped - coul) + coul


def _g_factor_slice(Rij, s):
    lo = s * _EPS
    planes = [Rij[lo:lo + _EPS, c].reshape(_SLICE_ROWS, 128) for c in range(3)]
    spec = pl.BlockSpec((_RB, 128), lambda i: (i, 0))
    out = pl.pallas_call(
        _g_body,
        grid=(_SLICE_ROWS // _RB,),
        in_specs=[spec, spec, spec],
        out_specs=spec,
        out_shape=jax.ShapeDtypeStruct((_SLICE_ROWS, 128), jnp.float32),
    )(*planes)
    return out.reshape(_EPS)


_NC, _NS, _L = 2, 16, 16
_NW = _NC * _NS
_EPW = _EPS // _NW
_CH = 2000
_NCHUNK = _EPW // _CH
_MB = 1008
_QMASK = np.int32(-1024)
_MMASK = np.int32(1023)
_QSCALE = np.float32(np.sqrt(0.5))


def _sc_scatter_body(sbase, packed_hbm, idxi_hbm, idxj_hbm, g_hbm, out_hbm,
                     tab, bins, bi0, bi1, bj0, bj1, bg0, bg1, sem0, sem1):
    wid = lax.axis_index("s") * _NC + lax.axis_index("c")
    base = wid * _EPW
    bi = (bi0, bi1)
    bj = (bj0, bj1)
    bg = (bg0, bg1)
    sems = (sem0, sem1)

    pltpu.sync_copy(packed_hbm, tab)

    zz = jnp.zeros((_L,), jnp.float32)

    def zero_body(c, carry):
        s = pl.ds(c * _L, _L)
        for r in range(_L):
            bins[r, s] = zz
        return carry

    lax.fori_loop(0, _MB // _L, zero_body, None)

    rows = lax.broadcasted_iota(jnp.int32, (_L,), 0)

    def start(c, slot):
        off = base + c * _CH
        pltpu.async_copy(idxi_hbm.at[pl.ds(sbase + off, _CH)], bi[slot],
                         sems[slot])
        pltpu.async_copy(idxj_hbm.at[pl.ds(sbase + off, _CH)], bj[slot],
                         sems[slot])
        pltpu.async_copy(g_hbm.at[pl.ds(off, _CH)], bg[slot], sems[slot])

    def drain(slot):
        pltpu.make_async_copy(idxi_hbm.at[pl.ds(0, _CH)], bi[slot],
                              sems[slot]).wait()
        pltpu.make_async_copy(idxj_hbm.at[pl.ds(0, _CH)], bj[slot],
                              sems[slot]).wait()
        pltpu.make_async_copy(g_hbm.at[pl.ds(0, _CH)], bg[slot],
                              sems[slot]).wait()

    def compute(slot):
        @plsc.parallel_loop(0, _CH, _L, unroll=4)
        def _(i):
            s = pl.ds(i, _L)
            vi = bi[slot][s]
            vj = bj[slot][s]
            gv = bg[slot][s]
            pi = plsc.load_gather(tab, [vi])
            pj = plsc.load_gather(tab, [vj])
            qi = plsc.bitcast(pi & _QMASK, jnp.float32)
            qj = plsc.bitcast(pj & _QMASK, jnp.float32)
            m = pi & _MMASK
            plsc.addupdate_scatter(bins, [rows, m], qi * qj * gv)

    start(0, 0)

    def ring_body(h, carry):
        c0 = 2 * h
        start(c0 + 1, 1)
        drain(0)
        compute(0)

        @pl.when(c0 + 2 < _NCHUNK)
        def _():
            start(c0 + 2, 0)

        drain(1)
        compute(1)
        return carry

    lax.fori_loop(0, _NCHUNK // 2, ring_body, None)

    def red_body(c, carry):
        s = pl.ds(c * _L, _L)
        acc = bins[0, s]
        for r in range(1, _L):
            acc = acc + bins[r, s]
        bins[0, s] = acc
        return carry

    lax.fori_loop(0, _MB // _L, red_body, None)

    pltpu.sync_copy(bins.at[0], out_hbm.at[wid])


@functools.cache
def _sc_scatter_kernel(sbase):
    mesh = plsc.VectorSubcoreMesh(core_axis_name="c", subcore_axis_name="s",
                                  num_cores=_NC, num_subcores=_NS)
    return pl.kernel(
        functools.partial(_sc_scatter_body, sbase),
        out_type=jax.ShapeDtypeStruct((_NW, _MB), jnp.float32),
        mesh=mesh,
        compiler_params=pltpu.CompilerParams(needs_layout_passes=False),
        scratch_types=[
            pltpu.VMEM((N_ATOMS,), jnp.int32),
            pltpu.VMEM((_L, _MB), jnp.float32),
            pltpu.VMEM((_CH,), jnp.int32),
            pltpu.VMEM((_CH,), jnp.int32),
            pltpu.VMEM((_CH,), jnp.int32),
            pltpu.VMEM((_CH,), jnp.int32),
            pltpu.VMEM((_CH,), jnp.float32),
            pltpu.VMEM((_CH,), jnp.float32),
            pltpu.SemaphoreType.DMA,
            pltpu.SemaphoreType.DMA,
        ],
    )


def kernel(Z, partial_charges, Rij, idx_i, idx_j, idx_m):
    q = jnp.squeeze(partial_charges, -1) * _QSCALE
    qbits = lax.bitcast_convert_type(q, jnp.int32)
    packed = ((qbits + 512) & _QMASK) | idx_m.astype(jnp.int32)
    idx_i = idx_i.astype(jnp.int32)
    idx_j = idx_j.astype(jnp.int32)
    partials = []
    for s in range(_NSLICE):
        g = _g_factor_slice(Rij, s)
        partials.append(_sc_scatter_kernel(s * _EPS)(packed, idx_i, idx_j, g))
    total = partials[0]
    for p in partials[1:]:
        total = total + p
    return jnp.sum(total, axis=0)[:N_MOL]
